# baseline (speedup 1.0000x reference)
import jax
import jax.numpy as jnp
from jax.experimental import pallas as pl

_BT = 512


def _router_block(x_ref, w_ref, b_ref, o_ref):
    logits = jax.lax.dot_general(
        x_ref[...], w_ref[...],
        dimension_numbers=(((1,), (1,)), ((), ())),
        preferred_element_type=jnp.float32) + b_ref[...]
    m = jnp.max(logits, axis=-1, keepdims=True)
    e = jnp.exp(logits - m)
    o_ref[...] = e * (1.0 / jnp.sum(e, axis=-1, keepdims=True))


def kernel(x, W, b):
    nt, h = x.shape
    ne = W.shape[0]
    b2 = b.reshape(1, ne)
    grid = (nt // _BT,)
    return pl.pallas_call(
        _router_block,
        grid=grid,
        in_specs=[
            pl.BlockSpec((_BT, h), lambda i: (i, 0)),
            pl.BlockSpec((ne, h), lambda i: (0, 0)),
            pl.BlockSpec((1, ne), lambda i: (0, 0)),
        ],
        out_specs=pl.BlockSpec((_BT, ne), lambda i: (i, 0)),
        out_shape=jax.ShapeDtypeStruct((nt, ne), jnp.float32),
    )(x, W, b2)


# speedup vs baseline: 1.0154x; 1.0154x over previous
import jax
import jax.numpy as jnp
from jax.experimental import pallas as pl

_BT = 512
_SPLIT = 2


def _router_block(xa_ref, xb_ref, w_ref, b_ref, o_ref):
    w = w_ref[...]
    b = b_ref[...]
    for k, x_ref in enumerate((xa_ref, xb_ref)):
        logits = jax.lax.dot_general(
            x_ref[...], w,
            dimension_numbers=(((1,), (1,)), ((), ())),
            preferred_element_type=jnp.float32) + b
        m = jnp.max(logits, axis=-1, keepdims=True)
        e = jnp.exp(logits - m)
        o_ref[k * _BT:(k + 1) * _BT, :] = (
            e * (1.0 / jnp.sum(e, axis=-1, keepdims=True)))


def kernel(x, W, b):
    nt, h = x.shape
    ne = W.shape[0]
    b2 = b.reshape(1, ne)
    grid = (nt // (_BT * _SPLIT),)
    return pl.pallas_call(
        _router_block,
        grid=grid,
        in_specs=[
            pl.BlockSpec((_BT, h), lambda i: (2 * i, 0)),
            pl.BlockSpec((_BT, h), lambda i: (2 * i + 1, 0)),
            pl.BlockSpec((ne, h), lambda i: (0, 0)),
            pl.BlockSpec((1, ne), lambda i: (0, 0)),
        ],
        out_specs=pl.BlockSpec((_BT * _SPLIT, ne), lambda i: (i, 0)),
        out_shape=jax.ShapeDtypeStruct((nt, ne), jnp.float32),
    )(x, x, W, b2)
